# baseline (speedup 1.0000x reference)
import functools

import jax
import jax.numpy as jnp
from jax import lax
from jax.experimental import pallas as pl
from jax.experimental.pallas import tpu as pltpu
from jax.experimental.pallas import tpu_sc as plsc

N = 10000
E = 160000
D = 256
DH = D // 2
C = float(D)
MIN_NORM = 1e-15
SQRT_C = C ** 0.5
MAXNORM = (1.0 - 4e-3) / SQRT_C

NS = 16
EB = 80
EPT = E // NS
SUP = EPT // EB
NB = 3
NOUT = (SUP // NB) * NB
RQ = 624


def _make_spmm():
    mesh = plsc.VectorSubcoreMesh(
        core_axis_name="c", subcore_axis_name="s", num_cores=2)

    @functools.partial(
        pl.kernel,
        out_type=jax.ShapeDtypeStruct((2 * N, DH), jnp.float32),
        mesh=mesh,
        scratch_types=[
            pltpu.VMEM((EPT,), jnp.int32),
            pltpu.VMEM((NB, EB, DH), jnp.float32),
            pltpu.VMEM((NB, EB), jnp.int32),
            pltpu.VMEM((NB, EB), jnp.float32),
            pltpu.VMEM_SHARED((N, DH), jnp.float32),
            pltpu.SemaphoreType.DMA((NB,)),
        ],
    )
    def spmm(x_hbm, col_hbm, row_hbm, w_hbm, z_hbm, out_hbm,
             cols_v, rows3_v, rids_v, wslt_v, ysp, gsem):
        c = lax.axis_index("c")
        s = lax.axis_index("s")
        hb0 = s * EPT

        pltpu.sync_copy(col_hbm.at[pl.ds(hb0, EPT)], cols_v)

        def xform_body(i, _):
            sl = pl.ds(i * 16, 16)
            cols_v[sl] = cols_v[sl] * 2 + c
            return 0
        lax.fori_loop(0, EPT // 16, xform_body, 0)

        pltpu.sync_copy(z_hbm, ysp.at[pl.ds(s * RQ, RQ)])

        @pl.when(s == NS - 1)
        def _():
            pltpu.sync_copy(z_hbm.at[pl.ds(0, 16)],
                            ysp.at[pl.ds(NS * RQ, 16)])

        def fire(k, b):
            base = k * EB
            pltpu.async_copy(row_hbm.at[pl.ds(hb0 + base, EB)],
                             rids_v.at[b], gsem.at[b])
            pltpu.async_copy(w_hbm.at[pl.ds(hb0 + base, EB)],
                             wslt_v.at[b], gsem.at[b])
            pltpu.async_copy(x_hbm.at[cols_v.at[pl.ds(base, EB)]],
                             rows3_v.at[b], gsem.at[b])

        def drain(k, b):
            base = k * EB
            pltpu.make_async_copy(row_hbm.at[pl.ds(hb0 + base, EB)],
                                  rids_v.at[b], gsem.at[b]).wait()
            pltpu.make_async_copy(w_hbm.at[pl.ds(hb0 + base, EB)],
                                  wslt_v.at[b], gsem.at[b]).wait()
            pltpu.make_async_copy(x_hbm.at[cols_v.at[pl.ds(base, EB)]],
                                  rows3_v.at[b], gsem.at[b]).wait()

        for b in range(NB):
            fire(b, b)

        plsc.subcore_barrier()

        def step(k, _):
            b = lax.rem(k, NB)
            drain(k, b)

            def scale_body(gr, _):
                wg = wslt_v[b, pl.ds(gr * 16, 16)]
                for j in range(16):
                    we = wg[j]
                    e = gr * 16 + j
                    for q in range(DH // 16):
                        sl = pl.ds(q * 16, 16)
                        rows3_v[b, e, sl] = rows3_v[b, e, sl] * we
                return 0
            lax.fori_loop(0, EB // 16, scale_body, 0, unroll=True)

            pltpu.sync_copy(rows3_v.at[b], ysp.at[rids_v.at[b]], add=True)

            @pl.when(k + NB < SUP)
            def _():
                fire(k + NB, b)
            return 0

        lax.fori_loop(0, SUP, step, 0)
        plsc.subcore_barrier()

        pltpu.sync_copy(ysp.at[pl.ds(s * RQ, RQ)],
                        out_hbm.at[pl.ds(c * N + s * RQ, RQ)])

        @pl.when(s == NS - 1)
        def _():
            pltpu.sync_copy(ysp.at[pl.ds(NS * RQ, 16)],
                            out_hbm.at[pl.ds(c * N + NS * RQ, 16)])

    return spmm


_spmm_cache = []


def _get_spmm():
    if not _spmm_cache:
        _spmm_cache.append(_make_spmm())
    return _spmm_cache[0]


BR = 1000
NBR = N // BR


def _rownorm(x):
    return jnp.sqrt(jnp.clip(jnp.sum(x * x, axis=-1, keepdims=True),
                             MIN_NORM * MIN_NORM, None))


def _clipnorm(n):
    return jnp.clip(n, MIN_NORM, None)


def _s1_body(y0_ref, y1_ref, w_ref, bias_ref, h_ref, acc_ref):
    i = pl.program_id(0)
    w = w_ref[...]
    h = lax.dot_general(y0_ref[...], w[:, :DH], (((1,), (1,)), ((), ())),
                        preferred_element_type=jnp.float32)
    h = h + lax.dot_general(y1_ref[...], w[:, DH:], (((1,), (1,)), ((), ())),
                            preferred_element_type=jnp.float32)

    un = _clipnorm(_rownorm(h))
    e = jnp.tanh(SQRT_C * un) * h / (SQRT_C * un)
    ne = _clipnorm(_rownorm(e))
    e = jnp.where(ne > MAXNORM, e / ne * MAXNORM, e)

    b = bias_ref[0, 0]
    bn = jnp.clip(jnp.abs(b), MIN_NORM, None)
    eb = jnp.tanh(SQRT_C * bn) * b / (SQRT_C * bn)
    nb = jnp.clip(jnp.abs(eb), MIN_NORM, None)
    vb = jnp.where(nb > MAXNORM, eb / nb * MAXNORM, eb)

    x2 = jnp.sum(e * e, axis=-1, keepdims=True)
    y2 = vb * vb
    xy = vb * jnp.sum(e, axis=-1, keepdims=True)
    num = (1.0 + 2.0 * C * xy + C * y2) * e + (1.0 - C * x2) * vb
    den = 1.0 + 2.0 * C * xy + C * C * x2 * y2
    m = num / jnp.clip(den, MIN_NORM, None)

    nm = _clipnorm(_rownorm(m))
    r = jnp.where(nm > MAXNORM, m / nm * MAXNORM, m)
    pn = _clipnorm(_rownorm(r))
    sarg = jnp.clip(SQRT_C * pn, -1.0 + 1e-7, 1.0 - 1e-7)
    atanh = 0.5 * jnp.log((1.0 + sarg) / (1.0 - sarg))
    hl = atanh * r / (SQRT_C * pn)

    h_ref[...] = hl

    @pl.when(i == 0)
    def _():
        acc_ref[...] = jnp.zeros_like(acc_ref)

    ssum = jnp.sum(hl, axis=0, keepdims=True)
    ssq = jnp.sum(hl * hl, axis=0, keepdims=True)
    upd = jnp.concatenate(
        [ssum, ssq, jnp.zeros((6, D), jnp.float32)], axis=0)
    acc_ref[...] = acc_ref[...] + upd


def _stage1(y01, W, bias2d):
    return pl.pallas_call(
        _s1_body,
        grid=(NBR,),
        in_specs=[
            pl.BlockSpec((BR, DH), lambda i: (i, 0)),
            pl.BlockSpec((BR, DH), lambda i: (NBR + i, 0)),
            pl.BlockSpec((D, D), lambda i: (0, 0)),
            pl.BlockSpec((1, 1), lambda i: (0, 0)),
        ],
        out_specs=[
            pl.BlockSpec((BR, D), lambda i: (i, 0)),
            pl.BlockSpec((8, D), lambda i: (0, 0)),
        ],
        out_shape=[
            jax.ShapeDtypeStruct((N, D), jnp.float32),
            jax.ShapeDtypeStruct((8, D), jnp.float32),
        ],
        compiler_params=pltpu.CompilerParams(
            dimension_semantics=("arbitrary",)),
    )(y01, y01, W, bias2d)


def _s3_body(h_ref, acc_ref, gamma_ref, beta_ref, out_ref):
    h = h_ref[...]
    mean = acc_ref[0:1, :] * (1.0 / N)
    ex2 = acc_ref[1:2, :] * (1.0 / N)
    var = ex2 - mean * mean
    xn = (h - mean) / jnp.sqrt(var + 1e-5) * gamma_ref[...] + beta_ref[...]
    out_ref[...] = h + jnp.maximum(xn, 0.0)


def _stage3(h, acc, gamma2d, beta2d):
    return pl.pallas_call(
        _s3_body,
        grid=(NBR,),
        in_specs=[
            pl.BlockSpec((BR, D), lambda i: (i, 0)),
            pl.BlockSpec((8, D), lambda i: (0, 0)),
            pl.BlockSpec((1, D), lambda i: (0, 0)),
            pl.BlockSpec((1, D), lambda i: (0, 0)),
        ],
        out_specs=pl.BlockSpec((BR, D), lambda i: (i, 0)),
        out_shape=jax.ShapeDtypeStruct((N, D), jnp.float32),
        compiler_params=pltpu.CompilerParams(
            dimension_semantics=("arbitrary",)),
    )(h, acc, gamma2d, beta2d)


def kernel(x, edge_index, edge_weight, W, bias, gamma, beta):
    row = edge_index[0]
    col = edge_index[1]
    x2 = x.reshape(2 * N, DH)
    zrows = jnp.zeros((RQ, DH), jnp.float32)
    y01 = _get_spmm()(x2, col, row, edge_weight, zrows)
    h, acc = _stage1(y01, W, bias.reshape(1, 1))
    out = _stage3(h, acc, gamma.reshape(1, D), beta.reshape(1, D))
    return out


# speedup vs baseline: 1.0443x; 1.0443x over previous
import functools

import jax
import jax.numpy as jnp
from jax import lax
from jax.experimental import pallas as pl
from jax.experimental.pallas import tpu as pltpu
from jax.experimental.pallas import tpu_sc as plsc

N = 10000
E = 160000
D = 256
DH = D // 2
C = float(D)
MIN_NORM = 1e-15
SQRT_C = C ** 0.5
MAXNORM = (1.0 - 4e-3) / SQRT_C

NS = 16
EB = 80
EPT = E // NS
SUP = EPT // EB
NI = 8
NG = 2
RQ = 624


def _make_spmm():
    mesh = plsc.VectorSubcoreMesh(
        core_axis_name="c", subcore_axis_name="s", num_cores=2)

    @functools.partial(
        pl.kernel,
        out_type=jax.ShapeDtypeStruct((2 * N, DH), jnp.float32),
        mesh=mesh,
        scratch_types=[
            pltpu.VMEM((NI, EB), jnp.int32),
            pltpu.VMEM((NI, EB), jnp.int32),
            pltpu.VMEM((NI, EB), jnp.float32),
            pltpu.VMEM((NG, EB, DH), jnp.float32),
            pltpu.VMEM((NG, EB, DH), jnp.float32),
            pltpu.VMEM_SHARED((N, DH), jnp.float32),
            pltpu.SemaphoreType.DMA((NI,)),
            pltpu.SemaphoreType.DMA((NG,)),
            pltpu.SemaphoreType.DMA((NG,)),
        ],
    )
    def spmm(x_hbm, col_hbm, row_hbm, w_hbm, z_hbm, out_hbm,
             cidx_v, rids_v, wslt_v, grow_v, sbuf_v, ysp,
             isem, gsem, ssem):
        c = lax.axis_index("c")
        s = lax.axis_index("s")
        hb0 = s * EPT

        pltpu.sync_copy(z_hbm, ysp.at[pl.ds(s * RQ, RQ)])

        @pl.when(s == NS - 1)
        def _():
            pltpu.sync_copy(z_hbm.at[pl.ds(0, 16)],
                            ysp.at[pl.ds(NS * RQ, 16)])

        def idx_copies(k, j):
            base = k * EB
            return (
                pltpu.make_async_copy(
                    col_hbm.at[pl.ds(c * E + hb0 + base, EB)],
                    cidx_v.at[j], isem.at[j]),
                pltpu.make_async_copy(
                    row_hbm.at[pl.ds(hb0 + base, EB)],
                    rids_v.at[j], isem.at[j]),
                pltpu.make_async_copy(
                    w_hbm.at[pl.ds(hb0 + base, EB)],
                    wslt_v.at[j], isem.at[j]),
            )

        def fire_idx(k, j):
            for d in idx_copies(k, j):
                d.start()

        def drain_idx(k, j):
            for d in idx_copies(k, j):
                d.wait()

        def gather_copy(j, b):
            return pltpu.make_async_copy(
                x_hbm.at[cidx_v.at[j]], grow_v.at[b], gsem.at[b])

        def scatter_copy(j, b):
            return pltpu.make_async_copy(
                sbuf_v.at[b], ysp.at[rids_v.at[j]], ssem.at[b])

        for k0 in range(NI - NG):
            fire_idx(k0, k0)
        for k0 in range(NG):
            drain_idx(k0, k0)
            gather_copy(k0, k0).start()

        plsc.subcore_barrier()

        def step(k, _):
            b = lax.rem(k, NG)
            j = lax.rem(k, NI)
            jn = lax.rem(k + NG, NI)

            gather_copy(j, b).wait()

            @pl.when(k + NI - NG < SUP)
            def _():
                fire_idx(k + NI - NG, lax.rem(k + NI - NG, NI))

            def scale_body(gr, _):
                wg = wslt_v[j, pl.ds(gr * 16, 16)]
                for jj in range(16):
                    we = wg[jj]
                    e = gr * 16 + jj
                    for q in range(DH // 16):
                        sl = pl.ds(q * 16, 16)
                        sbuf_v[b, e, sl] = grow_v[b, e, sl] * we
                return 0
            lax.fori_loop(0, EB // 16, scale_body, 0, unroll=True)

            @pl.when(k >= 1)
            def _():
                scatter_copy(lax.rem(k + NI - 1, NI), 1 - b).wait()

            scatter_copy(j, b).start(add=True)

            @pl.when(k + NG < SUP)
            def _():
                drain_idx(k + NG, jn)
                gather_copy(jn, b).start()
            return 0

        lax.fori_loop(0, SUP, step, 0)

        scatter_copy((SUP - 1) % NI, (SUP - 1) % NG).wait()

        plsc.subcore_barrier()

        pltpu.sync_copy(ysp.at[pl.ds(s * RQ, RQ)],
                        out_hbm.at[pl.ds(c * N + s * RQ, RQ)])

        @pl.when(s == NS - 1)
        def _():
            pltpu.sync_copy(ysp.at[pl.ds(NS * RQ, 16)],
                            out_hbm.at[pl.ds(c * N + NS * RQ, 16)])

    return spmm


_spmm_cache = []


def _get_spmm():
    if not _spmm_cache:
        _spmm_cache.append(_make_spmm())
    return _spmm_cache[0]


BR = 1000
NBR = N // BR


def _sc_body(y0_ref, y1_ref, w_ref, bias_ref, h_ref, acc_ref):
    i = pl.program_id(0)
    w = w_ref[...]
    h = lax.dot_general(y0_ref[...], w[:, :DH], (((1,), (1,)), ((), ())),
                        preferred_element_type=jnp.float32)
    h = h + lax.dot_general(y1_ref[...], w[:, DH:], (((1,), (1,)), ((), ())),
                            preferred_element_type=jnp.float32)

    sumsq = jnp.sum(h * h, axis=-1, keepdims=True)
    rowsum = jnp.sum(h, axis=-1, keepdims=True)

    un = jnp.clip(jnp.sqrt(jnp.clip(sumsq, MIN_NORM * MIN_NORM, None)),
                  MIN_NORM, None)
    z = SQRT_C * un
    t = jnp.tanh(z)
    alpha = t / z
    ne = jnp.clip(t / SQRT_C, MIN_NORM, None)
    p1 = jnp.where(ne > MAXNORM, MAXNORM / ne, 1.0)
    a = alpha * p1
    xnrm = jnp.minimum(ne, MAXNORM)
    x2 = xnrm * xnrm
    se = a * rowsum

    b = bias_ref[0, 0]
    bn = jnp.clip(jnp.abs(b), MIN_NORM, None)
    eb = jnp.tanh(SQRT_C * bn) * b / (SQRT_C * bn)
    nb = jnp.clip(jnp.abs(eb), MIN_NORM, None)
    vb = jnp.where(nb > MAXNORM, eb / nb * MAXNORM, eb)

    xy = vb * se
    y2 = vb * vb
    A = 1.0 + 2.0 * C * xy + C * y2
    B = 1.0 - C * x2
    den = jnp.clip(1.0 + 2.0 * C * xy + C * C * x2 * y2, MIN_NORM, None)

    sq_m = (A * A * x2 + 2.0 * A * B * vb * se + B * B * (D * (vb * vb)))
    sq_m = sq_m / (den * den)
    nm = jnp.clip(jnp.sqrt(jnp.clip(sq_m, MIN_NORM * MIN_NORM, None)),
                  MIN_NORM, None)
    p2 = jnp.where(nm > MAXNORM, MAXNORM / nm, 1.0)
    pn = jnp.clip(p2 * nm, MIN_NORM, None)

    sarg = jnp.clip(SQRT_C * pn, -1.0 + 1e-7, 1.0 - 1e-7)
    atanh = 0.5 * jnp.log((1.0 + sarg) / (1.0 - sarg))
    delta = atanh / (SQRT_C * pn)

    P = delta * p2 * A * a / den
    Q = delta * p2 * B * vb / den

    hl = P * h + Q
    h_ref[...] = hl

    @pl.when(i == 0)
    def _():
        acc_ref[...] = jnp.zeros_like(acc_ref)

    ssum = jnp.sum(hl, axis=0, keepdims=True)
    ssq = jnp.sum(hl * hl, axis=0, keepdims=True)
    upd = jnp.concatenate(
        [ssum, ssq, jnp.zeros((6, D), jnp.float32)], axis=0)
    acc_ref[...] = acc_ref[...] + upd


def _stage_c(y01, W, bias2d):
    return pl.pallas_call(
        _sc_body,
        grid=(NBR,),
        in_specs=[
            pl.BlockSpec((BR, DH), lambda i: (i, 0)),
            pl.BlockSpec((BR, DH), lambda i: (NBR + i, 0)),
            pl.BlockSpec((D, D), lambda i: (0, 0)),
            pl.BlockSpec((1, 1), lambda i: (0, 0)),
        ],
        out_specs=[
            pl.BlockSpec((BR, D), lambda i: (i, 0)),
            pl.BlockSpec((8, D), lambda i: (0, 0)),
        ],
        out_shape=[
            jax.ShapeDtypeStruct((N, D), jnp.float32),
            jax.ShapeDtypeStruct((8, D), jnp.float32),
        ],
        compiler_params=pltpu.CompilerParams(
            dimension_semantics=("arbitrary",)),
    )(y01, y01, W, bias2d)


def _sd_body(h_ref, acc_ref, gamma_ref, beta_ref, out_ref):
    h = h_ref[...]
    mean = acc_ref[0:1, :] * (1.0 / N)
    ex2 = acc_ref[1:2, :] * (1.0 / N)
    var = ex2 - mean * mean
    xn = (h - mean) / jnp.sqrt(var + 1e-5) * gamma_ref[...] + beta_ref[...]
    out_ref[...] = h + jnp.maximum(xn, 0.0)


def _stage_d(h, acc, gamma2d, beta2d):
    return pl.pallas_call(
        _sd_body,
        grid=(NBR,),
        in_specs=[
            pl.BlockSpec((BR, D), lambda i: (i, 0)),
            pl.BlockSpec((8, D), lambda i: (0, 0)),
            pl.BlockSpec((1, D), lambda i: (0, 0)),
            pl.BlockSpec((1, D), lambda i: (0, 0)),
        ],
        out_specs=pl.BlockSpec((BR, D), lambda i: (i, 0)),
        out_shape=jax.ShapeDtypeStruct((N, D), jnp.float32),
        compiler_params=pltpu.CompilerParams(
            dimension_semantics=("arbitrary",)),
    )(h, acc, gamma2d, beta2d)


def kernel(x, edge_index, edge_weight, W, bias, gamma, beta):
    row = edge_index[0]
    col = edge_index[1]
    x2 = x.reshape(2 * N, DH)
    cols2 = jnp.concatenate([2 * col, 2 * col + 1])
    zrows = jnp.zeros((RQ, DH), jnp.float32)
    y01 = _get_spmm()(x2, cols2, row, edge_weight, zrows)
    h, acc = _stage_c(y01, W, bias.reshape(1, 1))
    out = _stage_d(h, acc, gamma.reshape(1, D), beta.reshape(1, D))
    return out
